# Initial kernel scaffold; baseline (speedup 1.0000x reference)
import jax
import jax.numpy as jnp
from jax.experimental import pallas as pl


def kernel(x, expert_outputs, rewards, W, b, baseline_value):
    raise NotImplementedError("write your pallas kernel here")


# speedup vs baseline: 1.0921x; 1.0921x over previous
import functools

import jax
import jax.numpy as jnp
from jax import lax
from jax.experimental import pallas as pl
from jax.experimental.pallas import tpu as pltpu
from jax.experimental.pallas import tpu_sc as plsc

N_TOK = 4096
D_MODEL = 1024
N_EXP = 8
BLK_T = 512
N_BLK = N_TOK // BLK_T

SC_CORES = 2
SC_SUBCORES = 16
N_WORKERS = SC_CORES * SC_SUBCORES
TOK_PER_W = N_TOK // N_WORKERS
CHUNK = 32
N_CHUNK = TOK_PER_W // CHUNK
VECS_PER_CHUNK = CHUNK * D_MODEL // 16


def _tc_route(x_ref, w_ref, b_ref, gum_ref, rew_ref, base_ref,
              g0_ref, g1_ref, aux_ref):
    pid = pl.program_id(0)
    logits = jnp.dot(x_ref[...], w_ref[...],
                     preferred_element_type=jnp.float32) + b_ref[...]
    m = jnp.max(logits, axis=-1, keepdims=True)
    ex = jnp.exp(logits - m)
    probs = ex / jnp.sum(ex, axis=-1, keepdims=True)
    lp = jnp.log(probs + 1e-9)
    scores = lp + gum_ref[...]
    lane = lax.broadcasted_iota(jnp.int32, (BLK_T, N_EXP), 1)
    m1 = jnp.max(scores, axis=-1, keepdims=True)
    i0 = jnp.min(jnp.where(scores == m1, lane, N_EXP), axis=-1, keepdims=True)
    sc2 = jnp.where(lane == i0, -jnp.inf, scores)
    m2 = jnp.max(sc2, axis=-1, keepdims=True)
    i1 = jnp.min(jnp.where(sc2 == m2, lane, N_EXP), axis=-1, keepdims=True)
    selmask = (lane == i0) | (lane == i1)
    sel = jnp.sum(jnp.where(selmask, lp, 0.0), axis=-1, keepdims=True)
    adv = rew_ref[0] - base_ref[0, 0]
    partial = jnp.sum(adv * sel)
    row = lax.broadcasted_iota(jnp.int32, (BLK_T, 1), 0) + pid * BLK_T
    g0_ref[0] = i0 * N_TOK + row
    g1_ref[0] = i1 * N_TOK + row

    @pl.when(pid == 0)
    def _():
        aux_ref[0, 0] = 0.0

    aux_ref[0, 0] += partial

    @pl.when(pid == pl.num_programs(0) - 1)
    def _():
        aux_ref[0, 0] = aux_ref[0, 0] * (-1.0 / N_TOK)


_route_call = pl.pallas_call(
    _tc_route,
    grid=(N_BLK,),
    in_specs=[
        pl.BlockSpec((BLK_T, D_MODEL), lambda i: (i, 0)),
        pl.BlockSpec((D_MODEL, N_EXP), lambda i: (0, 0)),
        pl.BlockSpec((1, N_EXP), lambda i: (0, 0)),
        pl.BlockSpec((BLK_T, N_EXP), lambda i: (i, 0)),
        pl.BlockSpec((1, BLK_T, 1), lambda i: (i, 0, 0)),
        pl.BlockSpec((1, 1), lambda i: (0, 0), memory_space=pltpu.SMEM),
    ],
    out_specs=[
        pl.BlockSpec((1, BLK_T, 1), lambda i: (i, 0, 0)),
        pl.BlockSpec((1, BLK_T, 1), lambda i: (i, 0, 0)),
        pl.BlockSpec((1, 1), lambda i: (0, 0), memory_space=pltpu.SMEM),
    ],
    out_shape=[
        jax.ShapeDtypeStruct((N_BLK, BLK_T, 1), jnp.int32),
        jax.ShapeDtypeStruct((N_BLK, BLK_T, 1), jnp.int32),
        jax.ShapeDtypeStruct((1, 1), jnp.float32),
    ],
)


@functools.partial(
    pl.kernel,
    out_type=jax.ShapeDtypeStruct((N_TOK, D_MODEL), jnp.float32),
    mesh=plsc.VectorSubcoreMesh(core_axis_name="c", subcore_axis_name="s",
                                num_cores=SC_CORES, num_subcores=SC_SUBCORES),
    scratch_types=[
        pltpu.VMEM((CHUNK,), jnp.int32),
        pltpu.VMEM((CHUNK,), jnp.int32),
        pltpu.VMEM((CHUNK, D_MODEL), jnp.float32),
        pltpu.VMEM((CHUNK, D_MODEL), jnp.float32),
        pltpu.SemaphoreType.DMA,
    ],
)
def _sc_combine(eo_hbm, g0_hbm, g1_hbm, out_hbm,
                idx0_v, idx1_v, rows0, rows1, sem):
    wid = lax.axis_index("s") * SC_CORES + lax.axis_index("c")
    base = wid * TOK_PER_W

    def chunk_body(c, carry):
        tok = base + c * CHUNK
        pltpu.sync_copy(g0_hbm.at[pl.ds(tok, CHUNK)], idx0_v)
        pltpu.sync_copy(g1_hbm.at[pl.ds(tok, CHUNK)], idx1_v)
        cp0 = pltpu.async_copy(eo_hbm.at[idx0_v], rows0, sem)
        cp1 = pltpu.async_copy(eo_hbm.at[idx1_v], rows1, sem)
        cp0.wait()
        cp1.wait()

        @plsc.parallel_loop(0, VECS_PER_CHUNK, unroll=8)
        def _(i):
            r = i >> 6
            col = (i & 63) * 16
            a = rows0[r, pl.ds(col, 16)]
            bv = rows1[r, pl.ds(col, 16)]
            rows0[r, pl.ds(col, 16)] = (a + bv) * 0.5

        pltpu.sync_copy(rows0, out_hbm.at[pl.ds(tok, CHUNK)])
        return carry

    lax.fori_loop(0, N_CHUNK, chunk_body, 0)


def kernel(x, expert_outputs, rewards, W, b, baseline_value):
    B, T, D = x.shape
    E = expert_outputs.shape[0]
    n = B * T
    u = jax.random.uniform(jax.random.key(1234), (B, T, E),
                           minval=1e-10, maxval=1.0)
    gum = (-jnp.log(-jnp.log(u))).reshape(n, E)

    g0, g1, aux = _route_call(
        x.reshape(n, D),
        W,
        b.reshape(1, E),
        gum,
        rewards.reshape(N_BLK, BLK_T, 1),
        baseline_value.reshape(1, 1),
    )

    out = _sc_combine(
        expert_outputs.reshape(E * n, D),
        g0.reshape(n),
        g1.reshape(n),
    )
    return out.reshape(B, T, D), aux[0, 0]


# speedup vs baseline: 1.7922x; 1.6410x over previous
import functools

import jax
import jax.numpy as jnp
import numpy as np
from jax import lax
from jax.experimental import pallas as pl
from jax.experimental.pallas import tpu as pltpu
from jax.experimental.pallas import tpu_sc as plsc

N_TOK = 4096
D_MODEL = 1024
N_EXP = 8
BLK_T = 1024
N_BLK = N_TOK // BLK_T

SC_CORES = 2
SC_SUBCORES = 16
N_WORKERS = SC_CORES * SC_SUBCORES
TOK_PER_W = N_TOK // N_WORKERS
CHUNK = 16
N_CHUNK = TOK_PER_W // CHUNK


def _threefry_uniform_t(seed: int, n: int, e: int,
                        minval: float, maxval: float) -> np.ndarray:
    old = np.seterr(over="ignore")
    k0 = np.uint32(np.uint64(seed) >> np.uint64(32))
    k1 = np.uint32(np.uint64(seed) & np.uint64(0xFFFFFFFF))
    i = np.arange(n * e, dtype=np.uint64)
    x0 = (i >> np.uint64(32)).astype(np.uint32)
    x1 = i.astype(np.uint32)

    def rotl(x, d):
        return (x << np.uint32(d)) | (x >> np.uint32(32 - d))

    ks = [k0, k1, k0 ^ k1 ^ np.uint32(0x1BD11BDA)]
    rot = [[13, 15, 26, 6], [17, 29, 16, 24]]
    x0 = x0 + k0
    x1 = x1 + k1
    for r in range(5):
        for d in rot[r % 2]:
            x0 = x0 + x1
            x1 = rotl(x1, d) ^ x0
        x0 = x0 + ks[(r + 1) % 3]
        x1 = x1 + ks[(r + 2) % 3] + np.uint32(r + 1)
    bits = x0 ^ x1
    fb = (bits >> np.uint32(9)) | np.float32(1.0).view(np.uint32)
    floats = fb.view(np.float32) - np.float32(1.0)
    u = floats * np.float32(maxval - minval) + np.float32(minval)
    u = np.maximum(np.float32(minval), u)
    return np.ascontiguousarray(u.reshape(n, e).T)


_U_T = _threefry_uniform_t(1234, N_TOK, N_EXP, 1e-10, 1.0)


def _tc_route(w_ref, x_ref, b_ref, gum_ref, rew_ref, base_ref,
              g0_ref, g1_ref, aux_ref):
    pid = pl.program_id(0)
    logits = lax.dot_general(
        w_ref[...], x_ref[...], (((0,), (1,)), ((), ())),
        preferred_element_type=jnp.float32) + b_ref[...]
    m = jnp.max(logits, axis=0, keepdims=True)
    ex = jnp.exp(logits - m)
    probs = ex / jnp.sum(ex, axis=0, keepdims=True)
    lp = jnp.log(probs + 1e-9)
    scores = lp + gum_ref[...]
    sub = lax.broadcasted_iota(jnp.int32, (N_EXP, BLK_T), 0)
    m1 = jnp.max(scores, axis=0, keepdims=True)
    i0 = jnp.min(jnp.where(scores == m1, sub, N_EXP), axis=0, keepdims=True)
    sc2 = jnp.where(sub == i0, -jnp.inf, scores)
    m2 = jnp.max(sc2, axis=0, keepdims=True)
    i1 = jnp.min(jnp.where(sc2 == m2, sub, N_EXP), axis=0, keepdims=True)
    selmask = (sub == i0) | (sub == i1)
    sel = jnp.sum(jnp.where(selmask, lp, 0.0), axis=0, keepdims=True)
    rew = rew_ref[pl.ds(pid // 2, 1), pl.ds((pid % 2) * BLK_T, BLK_T)]
    adv = rew - base_ref[0, 0]
    partial = jnp.sum(adv * sel)
    n_row = lax.broadcasted_iota(jnp.int32, (1, BLK_T), 1) + pid * BLK_T
    g0_ref[0] = jnp.reshape(i0 * N_TOK + n_row, (8, 128))
    g1_ref[0] = jnp.reshape(i1 * N_TOK + n_row, (8, 128))

    @pl.when(pid == 0)
    def _():
        aux_ref[0, 0] = 0.0

    aux_ref[0, 0] += partial

    @pl.when(pid == pl.num_programs(0) - 1)
    def _():
        aux_ref[0, 0] = aux_ref[0, 0] * (-1.0 / N_TOK)


_route_call = pl.pallas_call(
    _tc_route,
    grid=(N_BLK,),
    in_specs=[
        pl.BlockSpec((D_MODEL, N_EXP), lambda i: (0, 0)),
        pl.BlockSpec((BLK_T, D_MODEL), lambda i: (i, 0)),
        pl.BlockSpec((N_EXP, 1), lambda i: (0, 0)),
        pl.BlockSpec((N_EXP, BLK_T), lambda i: (0, i)),
        pl.BlockSpec((2, 2048), lambda i: (0, 0)),
        pl.BlockSpec((1, 1), lambda i: (0, 0), memory_space=pltpu.SMEM),
    ],
    out_specs=[
        pl.BlockSpec((1, 8, 128), lambda i: (i, 0, 0)),
        pl.BlockSpec((1, 8, 128), lambda i: (i, 0, 0)),
        pl.BlockSpec((1, 1), lambda i: (0, 0), memory_space=pltpu.SMEM),
    ],
    out_shape=[
        jax.ShapeDtypeStruct((N_BLK, 8, 128), jnp.int32),
        jax.ShapeDtypeStruct((N_BLK, 8, 128), jnp.int32),
        jax.ShapeDtypeStruct((1, 1), jnp.float32),
    ],
)


@functools.partial(
    pl.kernel,
    out_type=jax.ShapeDtypeStruct((N_TOK, D_MODEL), jnp.float32),
    mesh=plsc.VectorSubcoreMesh(core_axis_name="c", subcore_axis_name="s",
                                num_cores=SC_CORES, num_subcores=SC_SUBCORES),
    scratch_types=[
        pltpu.VMEM((TOK_PER_W,), jnp.int32),
        pltpu.VMEM((TOK_PER_W,), jnp.int32),
        pltpu.VMEM((CHUNK, D_MODEL), jnp.float32),
        pltpu.VMEM((CHUNK, D_MODEL), jnp.float32),
        pltpu.VMEM((CHUNK, D_MODEL), jnp.float32),
        pltpu.VMEM((CHUNK, D_MODEL), jnp.float32),
        pltpu.VMEM((CHUNK, D_MODEL), jnp.float32),
        pltpu.VMEM((CHUNK, D_MODEL), jnp.float32),
        pltpu.SemaphoreType.DMA,
        pltpu.SemaphoreType.DMA,
        pltpu.SemaphoreType.DMA,
        pltpu.SemaphoreType.DMA,
    ],
)
def _sc_combine(eo_hbm, g0_hbm, g1_hbm, out_hbm, idx0_v, idx1_v,
                r0a, r0b, r1a, r1b, oa, ob, gsa, gsb, wsa, wsb):
    wid = lax.axis_index("s") * SC_CORES + lax.axis_index("c")
    base = wid * TOK_PER_W
    pltpu.sync_copy(g0_hbm.at[pl.ds(base, TOK_PER_W)], idx0_v)
    pltpu.sync_copy(g1_hbm.at[pl.ds(base, TOK_PER_W)], idx1_v)
    rows0 = (r0a, r0b)
    rows1 = (r1a, r1b)
    obuf = (oa, ob)
    gsem = (gsa, gsb)
    wsem = (wsa, wsb)
    handles = {}
    wbh = {}

    def issue(c):
        b = c & 1
        handles[c] = (
            pltpu.async_copy(
                eo_hbm.at[idx0_v.at[pl.ds(c * CHUNK, CHUNK)]], rows0[b], gsem[b]),
            pltpu.async_copy(
                eo_hbm.at[idx1_v.at[pl.ds(c * CHUNK, CHUNK)]], rows1[b], gsem[b]),
        )

    issue(0)
    issue(1)
    for c in range(N_CHUNK):
        b = c & 1
        handles[c][0].wait()
        handles[c][1].wait()
        if c >= 2:
            wbh[c - 2].wait()
        a_ref, b_ref, o_ref = rows0[b], rows1[b], obuf[b]

        @plsc.parallel_loop(0, CHUNK * D_MODEL // 16, unroll=8)
        def _(i):
            r = i >> 6
            col = (i & 63) * 16
            o_ref[r, pl.ds(col, 16)] = (
                a_ref[r, pl.ds(col, 16)] + b_ref[r, pl.ds(col, 16)]) * 0.5

        wbh[c] = pltpu.async_copy(
            o_ref, out_hbm.at[pl.ds(base + c * CHUNK, CHUNK)], wsem[b])
        if c + 2 < N_CHUNK:
            issue(c + 2)
    wbh[N_CHUNK - 2].wait()
    wbh[N_CHUNK - 1].wait()


def kernel(x, expert_outputs, rewards, W, b, baseline_value):
    B, T, D = x.shape
    E = expert_outputs.shape[0]
    n = B * T
    gum_t = -jnp.log(-jnp.log(jnp.asarray(_U_T)))

    g0, g1, aux = _route_call(
        W,
        x.reshape(n, D),
        b.reshape(E, 1),
        gum_t,
        rewards,
        baseline_value.reshape(1, 1),
    )

    out = _sc_combine(
        expert_outputs.reshape(E * n, D),
        g0.reshape(n),
        g1.reshape(n),
    )
    return out.reshape(B, T, D), aux[0, 0]


# speedup vs baseline: 1.8310x; 1.0216x over previous
import functools

import jax
import jax.numpy as jnp
import numpy as np
from jax import lax
from jax.experimental import pallas as pl
from jax.experimental.pallas import tpu as pltpu
from jax.experimental.pallas import tpu_sc as plsc

N_TOK = 4096
D_MODEL = 1024
N_EXP = 8
BLK_T = 1024
N_BLK = N_TOK // BLK_T

SC_CORES = 2
SC_SUBCORES = 16
N_WORKERS = SC_CORES * SC_SUBCORES
TOK_PER_W = N_TOK // N_WORKERS
CHUNK = 8
N_CHUNK = TOK_PER_W // CHUNK
NBUF = 4


def _threefry_uniform_t(seed: int, n: int, e: int,
                        minval: float, maxval: float) -> np.ndarray:
    old = np.seterr(over="ignore")
    k0 = np.uint32(np.uint64(seed) >> np.uint64(32))
    k1 = np.uint32(np.uint64(seed) & np.uint64(0xFFFFFFFF))
    i = np.arange(n * e, dtype=np.uint64)
    x0 = (i >> np.uint64(32)).astype(np.uint32)
    x1 = i.astype(np.uint32)

    def rotl(x, d):
        return (x << np.uint32(d)) | (x >> np.uint32(32 - d))

    ks = [k0, k1, k0 ^ k1 ^ np.uint32(0x1BD11BDA)]
    rot = [[13, 15, 26, 6], [17, 29, 16, 24]]
    x0 = x0 + k0
    x1 = x1 + k1
    for r in range(5):
        for d in rot[r % 2]:
            x0 = x0 + x1
            x1 = rotl(x1, d) ^ x0
        x0 = x0 + ks[(r + 1) % 3]
        x1 = x1 + ks[(r + 2) % 3] + np.uint32(r + 1)
    bits = x0 ^ x1
    fb = (bits >> np.uint32(9)) | np.float32(1.0).view(np.uint32)
    floats = fb.view(np.float32) - np.float32(1.0)
    u = floats * np.float32(maxval - minval) + np.float32(minval)
    u = np.maximum(np.float32(minval), u)
    np.seterr(**old)
    return np.ascontiguousarray(u.reshape(n, e).T)


_U_T = _threefry_uniform_t(1234, N_TOK, N_EXP, 1e-10, 1.0)


def _tc_route(w_ref, x_ref, b_ref, gum_ref, rew_ref, base_ref,
              g0_ref, g1_ref, aux_ref):
    pid = pl.program_id(0)
    logits = lax.dot_general(
        w_ref[...], x_ref[...], (((0,), (1,)), ((), ())),
        preferred_element_type=jnp.float32)
    sub0 = lax.broadcasted_iota(jnp.int32, (N_EXP, BLK_T), 0)
    for e in range(N_EXP):
        logits = logits + jnp.where(sub0 == e, b_ref[0, e], 0.0)
    m = jnp.max(logits, axis=0, keepdims=True)
    ex = jnp.exp(logits - m)
    probs = ex / jnp.sum(ex, axis=0, keepdims=True)
    lp = jnp.log(probs + 1e-9)
    scores = lp + gum_ref[...]
    sub = lax.broadcasted_iota(jnp.int32, (N_EXP, BLK_T), 0)
    m1 = jnp.max(scores, axis=0, keepdims=True)
    i0 = jnp.min(jnp.where(scores == m1, sub, N_EXP), axis=0, keepdims=True)
    sc2 = jnp.where(sub == i0, -jnp.inf, scores)
    m2 = jnp.max(sc2, axis=0, keepdims=True)
    i1 = jnp.min(jnp.where(sc2 == m2, sub, N_EXP), axis=0, keepdims=True)
    selmask = (sub == i0) | (sub == i1)
    sel = jnp.sum(jnp.where(selmask, lp, 0.0), axis=0, keepdims=True)
    rew = rew_ref[pl.ds(pid // 2, 1), pl.ds((pid % 2) * BLK_T, BLK_T)]
    adv = rew - base_ref[0, 0]
    partial = jnp.sum(adv * sel)
    n_row = lax.broadcasted_iota(jnp.int32, (1, BLK_T), 1) + pid * BLK_T
    g0_ref[0] = jnp.reshape(i0 * N_TOK + n_row, (BLK_T // 128, 128))
    g1_ref[0] = jnp.reshape(i1 * N_TOK + n_row, (BLK_T // 128, 128))

    @pl.when(pid == 0)
    def _():
        aux_ref[0, 0] = 0.0

    aux_ref[0, 0] += partial

    @pl.when(pid == pl.num_programs(0) - 1)
    def _():
        aux_ref[0, 0] = aux_ref[0, 0] * (-1.0 / N_TOK)


_route_call = pl.pallas_call(
    _tc_route,
    grid=(N_BLK,),
    in_specs=[
        pl.BlockSpec((D_MODEL, N_EXP), lambda i: (0, 0)),
        pl.BlockSpec((BLK_T, D_MODEL), lambda i: (i, 0)),
        pl.BlockSpec((1, N_EXP), lambda i: (0, 0), memory_space=pltpu.SMEM),
        pl.BlockSpec((N_EXP, BLK_T), lambda i: (0, i)),
        pl.BlockSpec((2, 2048), lambda i: (0, 0)),
        pl.BlockSpec((1, 1), lambda i: (0, 0), memory_space=pltpu.SMEM),
    ],
    out_specs=[
        pl.BlockSpec((1, BLK_T // 128, 128), lambda i: (i, 0, 0)),
        pl.BlockSpec((1, BLK_T // 128, 128), lambda i: (i, 0, 0)),
        pl.BlockSpec((1, 1), lambda i: (0, 0), memory_space=pltpu.SMEM),
    ],
    out_shape=[
        jax.ShapeDtypeStruct((N_BLK, BLK_T // 128, 128), jnp.int32),
        jax.ShapeDtypeStruct((N_BLK, BLK_T // 128, 128), jnp.int32),
        jax.ShapeDtypeStruct((1, 1), jnp.float32),
    ],
)


@functools.partial(
    pl.kernel,
    out_type=jax.ShapeDtypeStruct((N_TOK, D_MODEL), jnp.float32),
    mesh=plsc.VectorSubcoreMesh(core_axis_name="c", subcore_axis_name="s",
                                num_cores=SC_CORES, num_subcores=SC_SUBCORES),
    scratch_types=(
        [pltpu.VMEM((TOK_PER_W,), jnp.int32)] * 2
        + [pltpu.VMEM((CHUNK, D_MODEL), jnp.float32)] * (3 * NBUF)
        + [pltpu.SemaphoreType.DMA] * (2 * NBUF)
    ),
)
def _sc_combine(eo_hbm, g0_hbm, g1_hbm, out_hbm, idx0_v, idx1_v, *bufs):
    rows0 = bufs[0:NBUF]
    rows1 = bufs[NBUF:2 * NBUF]
    obuf = bufs[2 * NBUF:3 * NBUF]
    gsem = bufs[3 * NBUF:3 * NBUF + NBUF]
    wsem = bufs[3 * NBUF + NBUF:]
    wid = lax.axis_index("s") * SC_CORES + lax.axis_index("c")
    base = wid * TOK_PER_W
    pltpu.sync_copy(g0_hbm.at[pl.ds(base, TOK_PER_W)], idx0_v)
    pltpu.sync_copy(g1_hbm.at[pl.ds(base, TOK_PER_W)], idx1_v)
    handles = {}
    wbh = {}

    def issue(c):
        b = c % NBUF
        handles[c] = (
            pltpu.async_copy(
                eo_hbm.at[idx0_v.at[pl.ds(c * CHUNK, CHUNK)]], rows0[b], gsem[b]),
            pltpu.async_copy(
                eo_hbm.at[idx1_v.at[pl.ds(c * CHUNK, CHUNK)]], rows1[b], gsem[b]),
        )

    for c in range(NBUF):
        issue(c)
    for c in range(N_CHUNK):
        b = c % NBUF
        handles[c][0].wait()
        handles[c][1].wait()
        if c >= NBUF:
            wbh[c - NBUF].wait()
        a_ref, b_ref, o_ref = rows0[b], rows1[b], obuf[b]

        @plsc.parallel_loop(0, CHUNK * D_MODEL // 16, unroll=8)
        def _(i):
            r = i >> 6
            col = (i & 63) * 16
            o_ref[r, pl.ds(col, 16)] = (
                a_ref[r, pl.ds(col, 16)] + b_ref[r, pl.ds(col, 16)]) * 0.5

        wbh[c] = pltpu.async_copy(
            o_ref, out_hbm.at[pl.ds(base + c * CHUNK, CHUNK)], wsem[b])
        if c + NBUF < N_CHUNK:
            issue(c + NBUF)
    for c in range(N_CHUNK - NBUF, N_CHUNK):
        wbh[c].wait()


def kernel(x, expert_outputs, rewards, W, b, baseline_value):
    B, T, D = x.shape
    E = expert_outputs.shape[0]
    n = B * T
    gum_t = -jnp.log(-jnp.log(jnp.asarray(_U_T)))

    g0, g1, aux = _route_call(
        W,
        x.reshape(n, D),
        b.reshape(1, E),
        gum_t,
        rewards,
        baseline_value.reshape(1, 1),
    )

    out = _sc_combine(
        expert_outputs.reshape(E * n, D),
        g0.reshape(n),
        g1.reshape(n),
    )
    return out.reshape(B, T, D), aux[0, 0]


# speedup vs baseline: 1.8515x; 1.0112x over previous
import functools

import jax
import jax.numpy as jnp
import numpy as np
from jax import lax
from jax.experimental import pallas as pl
from jax.experimental.pallas import tpu as pltpu
from jax.experimental.pallas import tpu_sc as plsc

N_TOK = 4096
D_MODEL = 1024
N_EXP = 8
BLK_T = 1024
N_BLK = N_TOK // BLK_T

SC_CORES = 2
SC_SUBCORES = 16
N_WORKERS = SC_CORES * SC_SUBCORES
TOK_PER_W = N_TOK // N_WORKERS
CHUNK = 8
N_CHUNK = TOK_PER_W // CHUNK
NBUF = 4


def _threefry_uniform_t(seed: int, n: int, e: int,
                        minval: float, maxval: float) -> np.ndarray:
    old = np.seterr(over="ignore")
    k0 = np.uint32(np.uint64(seed) >> np.uint64(32))
    k1 = np.uint32(np.uint64(seed) & np.uint64(0xFFFFFFFF))
    i = np.arange(n * e, dtype=np.uint64)
    x0 = (i >> np.uint64(32)).astype(np.uint32)
    x1 = i.astype(np.uint32)

    def rotl(x, d):
        return (x << np.uint32(d)) | (x >> np.uint32(32 - d))

    ks = [k0, k1, k0 ^ k1 ^ np.uint32(0x1BD11BDA)]
    rot = [[13, 15, 26, 6], [17, 29, 16, 24]]
    x0 = x0 + k0
    x1 = x1 + k1
    for r in range(5):
        for d in rot[r % 2]:
            x0 = x0 + x1
            x1 = rotl(x1, d) ^ x0
        x0 = x0 + ks[(r + 1) % 3]
        x1 = x1 + ks[(r + 2) % 3] + np.uint32(r + 1)
    bits = x0 ^ x1
    fb = (bits >> np.uint32(9)) | np.float32(1.0).view(np.uint32)
    floats = fb.view(np.float32) - np.float32(1.0)
    u = floats * np.float32(maxval - minval) + np.float32(minval)
    u = np.maximum(np.float32(minval), u)
    np.seterr(**old)
    return np.ascontiguousarray(u.reshape(n, e).T)


_U_T = _threefry_uniform_t(1234, N_TOK, N_EXP, 1e-10, 1.0)


def _tc_route(w_ref, x_ref, b_ref, gum_ref, rew_ref, base_ref,
              g0_ref, g1_ref, aux_ref):
    pid = pl.program_id(0)
    logits = lax.dot_general(
        w_ref[...], x_ref[...], (((0,), (1,)), ((), ())),
        preferred_element_type=jnp.float32)
    sub0 = lax.broadcasted_iota(jnp.int32, (N_EXP, BLK_T), 0)
    for e in range(N_EXP):
        logits = logits + jnp.where(sub0 == e, b_ref[0, e], 0.0)
    m = jnp.max(logits, axis=0, keepdims=True)
    ex = jnp.exp(logits - m)
    probs = ex / jnp.sum(ex, axis=0, keepdims=True)
    lp = jnp.log(probs + 1e-9)
    scores = lp + gum_ref[...]
    sub = lax.broadcasted_iota(jnp.int32, (N_EXP, BLK_T), 0)
    m1 = jnp.max(scores, axis=0, keepdims=True)
    i0 = jnp.min(jnp.where(scores == m1, sub, N_EXP), axis=0, keepdims=True)
    sc2 = jnp.where(sub == i0, -jnp.inf, scores)
    m2 = jnp.max(sc2, axis=0, keepdims=True)
    i1 = jnp.min(jnp.where(sc2 == m2, sub, N_EXP), axis=0, keepdims=True)
    selmask = (sub == i0) | (sub == i1)
    sel = jnp.sum(jnp.where(selmask, lp, 0.0), axis=0, keepdims=True)
    rew = rew_ref[pl.ds(pid // 2, 1), pl.ds((pid % 2) * BLK_T, BLK_T)]
    adv = rew - base_ref[0, 0]
    partial = jnp.sum(adv * sel)
    n_row = lax.broadcasted_iota(jnp.int32, (1, BLK_T), 1) + pid * BLK_T
    g0_ref[0] = jnp.reshape(i0 * N_TOK + n_row, (BLK_T // 128, 128))
    g1_ref[0] = jnp.reshape(i1 * N_TOK + n_row, (BLK_T // 128, 128))

    @pl.when(pid == 0)
    def _():
        aux_ref[0, 0] = 0.0

    aux_ref[0, 0] += partial

    @pl.when(pid == pl.num_programs(0) - 1)
    def _():
        aux_ref[0, 0] = aux_ref[0, 0] * (-1.0 / N_TOK)


_route_call = pl.pallas_call(
    _tc_route,
    grid=(N_BLK,),
    in_specs=[
        pl.BlockSpec((D_MODEL, N_EXP), lambda i: (0, 0)),
        pl.BlockSpec((BLK_T, D_MODEL), lambda i: (i, 0)),
        pl.BlockSpec((1, N_EXP), lambda i: (0, 0), memory_space=pltpu.SMEM),
        pl.BlockSpec((N_EXP, BLK_T), lambda i: (0, i)),
        pl.BlockSpec((2, 2048), lambda i: (0, 0)),
        pl.BlockSpec((1, 1), lambda i: (0, 0), memory_space=pltpu.SMEM),
    ],
    out_specs=[
        pl.BlockSpec((1, BLK_T // 128, 128), lambda i: (i, 0, 0)),
        pl.BlockSpec((1, BLK_T // 128, 128), lambda i: (i, 0, 0)),
        pl.BlockSpec((1, 1), lambda i: (0, 0), memory_space=pltpu.SMEM),
    ],
    out_shape=[
        jax.ShapeDtypeStruct((N_BLK, BLK_T // 128, 128), jnp.int32),
        jax.ShapeDtypeStruct((N_BLK, BLK_T // 128, 128), jnp.int32),
        jax.ShapeDtypeStruct((1, 1), jnp.float32),
    ],
)


@functools.partial(
    pl.kernel,
    out_type=jax.ShapeDtypeStruct((N_TOK, D_MODEL), jnp.float32),
    mesh=plsc.VectorSubcoreMesh(core_axis_name="c", subcore_axis_name="s",
                                num_cores=SC_CORES, num_subcores=SC_SUBCORES),
    scratch_types=(
        [pltpu.VMEM((TOK_PER_W,), jnp.int32)] * 2
        + [pltpu.VMEM((CHUNK, D_MODEL), jnp.float32)] * (3 * NBUF)
        + [pltpu.SemaphoreType.DMA] * (2 * NBUF)
    ),
)
def _sc_combine(eo_hbm, g0_hbm, g1_hbm, out_hbm, idx0_v, idx1_v, *bufs):
    rows0 = bufs[0:NBUF]
    rows1 = bufs[NBUF:2 * NBUF]
    obuf = bufs[2 * NBUF:3 * NBUF]
    gsem = bufs[3 * NBUF:3 * NBUF + NBUF]
    wsem = bufs[3 * NBUF + NBUF:]
    wid = lax.axis_index("s") * SC_CORES + lax.axis_index("c")
    base = wid * TOK_PER_W
    pltpu.sync_copy(g0_hbm.at[pl.ds(base, TOK_PER_W)], idx0_v)
    pltpu.sync_copy(g1_hbm.at[pl.ds(base, TOK_PER_W)], idx1_v)
    handles = {}
    wbh = {}

    def issue(c):
        b = c % NBUF
        handles[c] = (
            pltpu.async_copy(
                eo_hbm.at[idx0_v.at[pl.ds(c * CHUNK, CHUNK)]], rows0[b], gsem[b]),
            pltpu.async_copy(
                eo_hbm.at[idx1_v.at[pl.ds(c * CHUNK, CHUNK)]], rows1[b], gsem[b]),
        )

    for c in range(NBUF - 1):
        issue(c)
    for c in range(N_CHUNK):
        b = c % NBUF
        handles[c][0].wait()
        handles[c][1].wait()
        if c >= NBUF:
            wbh[c - NBUF].wait()
        if c + NBUF - 1 < N_CHUNK:
            issue(c + NBUF - 1)
        a_ref, b_ref, o_ref = rows0[b], rows1[b], obuf[b]

        @plsc.parallel_loop(0, CHUNK * D_MODEL // 16, unroll=8)
        def _(i):
            r = i >> 6
            col = (i & 63) * 16
            o_ref[r, pl.ds(col, 16)] = (
                a_ref[r, pl.ds(col, 16)] + b_ref[r, pl.ds(col, 16)]) * 0.5

        wbh[c] = pltpu.async_copy(
            o_ref, out_hbm.at[pl.ds(base + c * CHUNK, CHUNK)], wsem[b])
    for c in range(N_CHUNK - NBUF, N_CHUNK):
        wbh[c].wait()


def kernel(x, expert_outputs, rewards, W, b, baseline_value):
    B, T, D = x.shape
    E = expert_outputs.shape[0]
    n = B * T
    gum_t = -jnp.log(-jnp.log(jnp.asarray(_U_T)))

    g0, g1, aux = _route_call(
        W,
        x.reshape(n, D),
        b.reshape(1, E),
        gum_t,
        rewards,
        baseline_value.reshape(1, 1),
    )

    out = _sc_combine(
        expert_outputs.reshape(E * n, D),
        g0.reshape(n),
        g1.reshape(n),
    )
    return out.reshape(B, T, D), aux[0, 0]
